# baseline (speedup 1.0000x reference)
import functools

import jax
import jax.numpy as jnp
import numpy as np
from jax import lax
from jax.experimental import pallas as pl
from jax.experimental.pallas import tpu as pltpu
from jax.experimental.pallas import tpu_sc as plsc

N_NODES = 10000
N_EDGES = 320000
D = 128
RBF = 16

NC = 2
NS = 16
NW = NC * NS
NH = 2
E_H = N_EDGES // NH
EPW = E_H // NW
C = 40
K = EPW // C
EB_F = EPW
N_PAD = 10240
ROWS_PER_TILE = N_PAD // NS

EB = 2560


def _sum_linear_kernel(p_ref, w_ref, b_ref, o_ref):
    s = p_ref[0, :N_NODES, :] + p_ref[1, :N_NODES, :]
    o_ref[...] = (
        jnp.dot(s, w_ref[...], preferred_element_type=jnp.float32) + b_ref[...]
    )


def _final_linear(p, w, b):
    return pl.pallas_call(
        _sum_linear_kernel,
        out_shape=jax.ShapeDtypeStruct((N_NODES, w.shape[1]), jnp.float32),
    )(p, w, b.reshape(1, -1))


def _pack_bf16_words(f):
    u = jax.lax.bitcast_convert_type(f.astype(jnp.bfloat16), jnp.uint16)
    lo = u[:, :64].astype(jnp.int32)
    hi = u[:, 64:].astype(jnp.int32)
    return jnp.bitwise_or(lo, jnp.left_shift(hi, 16))


def _col_perm():
    s = np.arange(D)
    return 32 * ((s % 64) // 16) + 16 * (s // 64) + (s % 16)


def _filter_body(rbf_ref, w1_ref, b1_ref, w2_ref, b2_ref):
    h = (
        jnp.dot(rbf_ref[...], w1_ref[...], preferred_element_type=jnp.float32)
        + b1_ref[...]
    )
    h = h * jax.nn.sigmoid(h)
    return (
        jnp.dot(h.astype(jnp.bfloat16), w2_ref[...],
                preferred_element_type=jnp.float32)
        + b2_ref[...]
    )


def _filter_kernel_a(x_ref, lw_ref, lb_ref, rbf_ref, w1_ref, b1_ref, w2_ref,
                     b2_ref, o_ref, ox_ref, oz_ref):
    o_ref[...] = _filter_body(rbf_ref, w1_ref, b1_ref, w2_ref, b2_ref)

    @pl.when(pl.program_id(0) < N_PAD // EB)
    def _():
        ox_ref[...] = (
            jnp.dot(x_ref[...], lw_ref[...],
                    preferred_element_type=jnp.float32)
            + lb_ref[...]
        )
        oz_ref[...] = jnp.zeros_like(oz_ref)


def _filter_kernel_b(rbf_ref, w1_ref, b1_ref, w2_ref, b2_ref, o_ref):
    o_ref[...] = _pack_bf16_words(
        _filter_body(rbf_ref, w1_ref, b1_ref, w2_ref, b2_ref))


def _filter_net(x, lin1_w, lin1_b, edge_rbf, fn1_w, fn1_b, fn2_w, fn2_b,
                half):
    w2 = fn2_w.astype(jnp.bfloat16)
    b2 = fn2_b
    grid = (NW,)
    fspecs = [
        pl.BlockSpec((EB_F, RBF), lambda i: (2 * i + half, 0)),
        pl.BlockSpec((RBF, D), lambda i: (0, 0)),
        pl.BlockSpec((1, D), lambda i: (0, 0)),
        pl.BlockSpec((D, D), lambda i: (0, 0)),
        pl.BlockSpec((1, D), lambda i: (0, 0)),
    ]
    fout = pl.BlockSpec((EB_F, D), lambda i: (i, 0))
    if half != 0:
        perm = _col_perm()
        fargs = (edge_rbf, fn1_w, fn1_b.reshape(1, D),
                 w2[:, perm], b2[perm].reshape(1, D))
        return pl.pallas_call(
            _filter_kernel_b,
            grid=grid,
            in_specs=fspecs,
            out_specs=pl.BlockSpec((EB_F, D // 2), lambda i: (i, 0)),
            out_shape=jax.ShapeDtypeStruct((E_H, D // 2), jnp.int32),
        )(*fargs)
    fargs = (edge_rbf, fn1_w, fn1_b.reshape(1, D), w2, b2.reshape(1, D))
    nx = N_PAD // EB
    nspec = pl.BlockSpec((EB, D), lambda i: (jnp.minimum(i, nx - 1), 0))
    return pl.pallas_call(
        _filter_kernel_a,
        grid=grid,
        in_specs=[
            nspec,
            pl.BlockSpec((D, D), lambda i: (0, 0)),
            pl.BlockSpec((1, D), lambda i: (0, 0)),
            *fspecs,
        ],
        out_specs=[fout, nspec, nspec],
        out_shape=[
            jax.ShapeDtypeStruct((E_H, D), jnp.float32),
            jax.ShapeDtypeStruct((N_NODES, D), jnp.float32),
            jax.ShapeDtypeStruct((N_PAD, D), jnp.float32),
        ],
    )(x, lin1_w, lin1_b.reshape(1, D), *fargs)


NBUF = 4
PF = 2
IRING = 8
UNROLL = IRING


FRING = 2


def _make_sc_body(half):
    return functools.partial(_sc_body, half)


def _sc_body(half, x1_hbm, idx_hbm, filt_hbm, zeros_hbm, out_hbm, idxv, *rest):
    gbufs = rest[0:NBUF]
    fbufs = rest[NBUF:NBUF + FRING]
    acc = rest[NBUF + FRING]
    o = NBUF + FRING + 1
    g_sems = rest[o:o + NBUF]
    f_sems = rest[o + NBUF:o + NBUF + FRING]
    s_sems = rest[o + NBUF + FRING:o + 2 * NBUF + FRING]
    i_sems = rest[o + 2 * NBUF + FRING:o + 2 * NBUF + FRING + IRING]
    j_sems = rest[o + 2 * NBUF + FRING + IRING:
                  o + 2 * NBUF + FRING + 2 * IRING]

    cid = lax.axis_index("c")
    sid = lax.axis_index("s")
    wid = sid * NC + cid
    base0 = wid * EPW

    if half == 0:
        init_src = zeros_hbm.at[pl.ds(sid * ROWS_PER_TILE, ROWS_PER_TILE)]
    else:
        init_src = zeros_hbm.at[cid, pl.ds(sid * ROWS_PER_TILE, ROWS_PER_TILE)]
    pltpu.sync_copy(
        init_src, acc.at[pl.ds(sid * ROWS_PER_TILE, ROWS_PER_TILE)])
    plsc.subcore_barrier()

    koff = half * K

    def idx_fire(i, q):
        pltpu.async_copy(
            idx_hbm.at[0, wid, koff + i], idxv.at[q, 0], i_sems[q])
        pltpu.async_copy(
            idx_hbm.at[1, wid, koff + i], idxv.at[q, 1], j_sems[q])

    def idx_wait(i, q):
        pltpu.make_async_copy(
            idx_hbm.at[0, wid, koff + i], idxv.at[q, 0], i_sems[q]).wait()
        pltpu.make_async_copy(
            idx_hbm.at[1, wid, koff + i], idxv.at[q, 1], j_sems[q]).wait()

    def g_fire(b, q):
        pltpu.async_copy(x1_hbm.at[idxv.at[q, 1]], gbufs[b], g_sems[b])

    def f_fire(i, fs):
        pltpu.async_copy(
            filt_hbm.at[pl.ds(base0 + i * C, 2 * C)], fbufs[fs], f_sems[fs])

    def f_wait(i, fs):
        pltpu.make_async_copy(
            filt_hbm.at[pl.ds(base0 + i * C, 2 * C)], fbufs[fs],
            f_sems[fs]).wait()

    def s_wait(b):
        pltpu.make_async_copy(gbufs[b], acc.at[idxv.at[0, 0]], s_sems[b]).wait()

    def unpack_words(w):
        lo = lax.bitcast_convert_type(jnp.left_shift(w, 16), jnp.float32)
        hi = lax.bitcast_convert_type(
            jnp.bitwise_and(w, jnp.int32(-65536)), jnp.float32)
        return lo, hi

    def proc(i, u):
        b = u % NBUF
        b2 = (u + PF) % NBUF
        q2 = (u + PF) % IRING
        q4 = (u + PF + 2) % IRING

        @pl.when(i + PF + 2 < K)
        def _():
            idx_fire(i + PF + 2, q4)

        @pl.when(i + PF < K)
        def _():
            @pl.when(i >= PF)
            def _():
                s_wait(b2)
            idx_wait(i + PF, q2)
            g_fire(b2, q2)
            if u % 2 == 0:
                f_fire(i + PF, ((u + PF) // 2) % FRING)

        pltpu.make_async_copy(
            x1_hbm.at[idxv.at[u, 1]], gbufs[b], g_sems[b]).wait()
        fs = (u // 2) % FRING
        if u % 2 == 0:
            f_wait(i, fs)

        gb, fb = gbufs[b], fbufs[fs]
        ro = (u % 2) * C

        def mul2(r, c2):
            for rr in range(2):
                rg = 2 * r + rr
                if half == 0:
                    for j in range(D // 16):
                        s = pl.ds(j * 16, 16)
                        gb[rg, s] = gb[rg, s] * fb[ro + rg, s]
                else:
                    for j in range(D // 32):
                        flo, fhi = unpack_words(fb[ro + rg, pl.ds(16 * j, 16)])
                        s0 = pl.ds(j * 32, 16)
                        s1 = pl.ds(j * 32 + 16, 16)
                        gb[rg, s0] = gb[rg, s0] * flo
                        gb[rg, s1] = gb[rg, s1] * fhi
            return c2

        lax.fori_loop(0, C // 2, mul2, 0)
        pltpu.async_copy(gb, acc.at[idxv.at[u, 0]], s_sems[b], add=True)

    for j in range(PF + 2):
        idx_fire(j, j)
    f_fire(0, 0)
    for j in range(PF):
        idx_wait(j, j)
        g_fire(j % NBUF, j)

    def outer(i2, carry):
        for u in range(UNROLL):
            proc(i2 * UNROLL + u, u)
        return carry

    n_outer = K // UNROLL
    lax.fori_loop(0, n_outer, outer, 0)
    for i in range(n_outer * UNROLL, K):
        proc(jnp.int32(i), i % UNROLL)
    for b in range(NBUF):
        s_wait(b)

    plsc.subcore_barrier()

    pltpu.sync_copy(
        acc.at[pl.ds(sid * ROWS_PER_TILE, ROWS_PER_TILE)],
        out_hbm.at[cid, pl.ds(sid * ROWS_PER_TILE, ROWS_PER_TILE)],
    )


def _sc_gather_mul_scatter(x1, edge_idx, filt, zeros, half):
    mesh = plsc.VectorSubcoreMesh(core_axis_name="c", subcore_axis_name="s")
    f = functools.partial(
        pl.kernel,
        mesh=mesh,
        out_type=jax.ShapeDtypeStruct((NC, N_PAD, D), jnp.float32),
        scratch_types=[
            pltpu.VMEM((IRING, 2, C), jnp.int32),
            *[pltpu.VMEM((C, D), jnp.float32) for _ in range(NBUF)],
            *[(pltpu.VMEM((2 * C, D), jnp.float32) if half == 0
               else pltpu.VMEM((2 * C, D // 2), jnp.int32))
              for _ in range(FRING)],
            pltpu.VMEM_SHARED((N_PAD, D), jnp.float32),
            *[pltpu.SemaphoreType.DMA
              for _ in range(2 * NBUF + FRING + 2 * IRING)],
        ],
    )(_make_sc_body(half))
    return f(x1, edge_idx, filt, zeros)


def kernel(x, edge_index, edge_rbf, lin1_w, lin1_b, lin2_w, lin2_b,
           fn1_w, fn1_b, fn2_w, fn2_b):
    ei = edge_index.astype(jnp.int32)
    filt_a, x1, zeros = _filter_net(
        x, lin1_w, lin1_b, edge_rbf, fn1_w, fn1_b, fn2_w, fn2_b, 0)
    filt_b = _filter_net(
        x, lin1_w, lin1_b, edge_rbf, fn1_w, fn1_b, fn2_w, fn2_b, 1)
    idx4 = ei.reshape(2, NW, NH * K, C)
    pa = _sc_gather_mul_scatter(x1, idx4, filt_a, zeros, 0)
    pb = _sc_gather_mul_scatter(x1, idx4, filt_b, pa, 1)
    return _final_linear(pb, lin2_w, lin2_b)


# speedup vs baseline: 1.0018x; 1.0018x over previous
import functools

import jax
import jax.numpy as jnp
import numpy as np
from jax import lax
from jax.experimental import pallas as pl
from jax.experimental.pallas import tpu as pltpu
from jax.experimental.pallas import tpu_sc as plsc

N_NODES = 10000
N_EDGES = 320000
D = 128
RBF = 16

NC = 2
NS = 16
NW = NC * NS
NH = 2
E_H = N_EDGES // NH
EPW = E_H // NW
C = 40
K = EPW // C
EB_F = EPW
N_PAD = 10240
ROWS_PER_TILE = N_PAD // NS

EB = 2560


def _sum_linear_kernel(p_ref, w_ref, b_ref, o_ref):
    s = p_ref[0, :N_NODES, :] + p_ref[1, :N_NODES, :]
    o_ref[...] = (
        jnp.dot(s, w_ref[...], preferred_element_type=jnp.float32) + b_ref[...]
    )


def _final_linear(p, w, b):
    return pl.pallas_call(
        _sum_linear_kernel,
        out_shape=jax.ShapeDtypeStruct((N_NODES, w.shape[1]), jnp.float32),
    )(p, w, b.reshape(1, -1))


def _pack_bf16_words(f):
    u = jax.lax.bitcast_convert_type(f.astype(jnp.bfloat16), jnp.uint16)
    lo = u[:, :64].astype(jnp.int32)
    hi = u[:, 64:].astype(jnp.int32)
    return jnp.bitwise_or(lo, jnp.left_shift(hi, 16))


def _col_perm():
    s = np.arange(D)
    return 32 * ((s % 64) // 16) + 16 * (s // 64) + (s % 16)


def _filter_body(rbf_ref, w1_ref, b1_ref, w2_ref, b2_ref):
    h = (
        jnp.dot(rbf_ref[...], w1_ref[...], preferred_element_type=jnp.float32)
        + b1_ref[...]
    )
    h = h * jax.nn.sigmoid(h)
    return (
        jnp.dot(h.astype(jnp.bfloat16), w2_ref[...],
                preferred_element_type=jnp.float32)
        + b2_ref[...]
    )


def _filter_kernel_a(x_ref, lw_ref, lb_ref, rbf_ref, w1_ref, b1_ref, w2_ref,
                     b2_ref, o_ref, ox_ref, oz_ref):
    o_ref[...] = _filter_body(rbf_ref, w1_ref, b1_ref, w2_ref, b2_ref)

    @pl.when(pl.program_id(0) < N_PAD // EB)
    def _():
        ox_ref[...] = (
            jnp.dot(x_ref[...], lw_ref[...],
                    preferred_element_type=jnp.float32)
            + lb_ref[...]
        )
        oz_ref[...] = jnp.zeros_like(oz_ref)


def _filter_kernel_b(rbf_ref, w1_ref, b1_ref, w2_ref, b2_ref, o_ref):
    o_ref[...] = _filter_body(rbf_ref, w1_ref, b1_ref, w2_ref, b2_ref)


def _filter_net(x, lin1_w, lin1_b, edge_rbf, fn1_w, fn1_b, fn2_w, fn2_b,
                half):
    w2 = fn2_w.astype(jnp.bfloat16)
    b2 = fn2_b
    grid = (NW,)
    fspecs = [
        pl.BlockSpec((EB_F, RBF), lambda i: (2 * i + half, 0)),
        pl.BlockSpec((RBF, D), lambda i: (0, 0)),
        pl.BlockSpec((1, D), lambda i: (0, 0)),
        pl.BlockSpec((D, D), lambda i: (0, 0)),
        pl.BlockSpec((1, D), lambda i: (0, 0)),
    ]
    fargs = (edge_rbf, fn1_w, fn1_b.reshape(1, D), w2, b2.reshape(1, D))
    fout = pl.BlockSpec((EB_F, D), lambda i: (i, 0))
    if half != 0:
        return pl.pallas_call(
            _filter_kernel_b,
            grid=grid,
            in_specs=fspecs,
            out_specs=fout,
            out_shape=jax.ShapeDtypeStruct((E_H, D), jnp.float32),
        )(*fargs)
    nx = N_PAD // EB
    nspec = pl.BlockSpec((EB, D), lambda i: (jnp.minimum(i, nx - 1), 0))
    return pl.pallas_call(
        _filter_kernel_a,
        grid=grid,
        in_specs=[
            nspec,
            pl.BlockSpec((D, D), lambda i: (0, 0)),
            pl.BlockSpec((1, D), lambda i: (0, 0)),
            *fspecs,
        ],
        out_specs=[fout, nspec, nspec],
        out_shape=[
            jax.ShapeDtypeStruct((E_H, D), jnp.float32),
            jax.ShapeDtypeStruct((N_NODES, D), jnp.float32),
            jax.ShapeDtypeStruct((N_PAD, D), jnp.float32),
        ],
    )(x, lin1_w, lin1_b.reshape(1, D), *fargs)


NBUF = 4
PF = 2
IRING = 8
UNROLL = IRING


FRING = 2


def _make_sc_body(half):
    return functools.partial(_sc_body, half)


def _sc_body(half, x1_hbm, idx_hbm, filt_hbm, zeros_hbm, out_hbm, idxv, *rest):
    gbufs = rest[0:NBUF]
    fbufs = rest[NBUF:NBUF + FRING]
    acc = rest[NBUF + FRING]
    o = NBUF + FRING + 1
    g_sems = rest[o:o + NBUF]
    f_sems = rest[o + NBUF:o + NBUF + FRING]
    s_sems = rest[o + NBUF + FRING:o + 2 * NBUF + FRING]
    i_sems = rest[o + 2 * NBUF + FRING:o + 2 * NBUF + FRING + IRING]
    j_sems = rest[o + 2 * NBUF + FRING + IRING:
                  o + 2 * NBUF + FRING + 2 * IRING]

    cid = lax.axis_index("c")
    sid = lax.axis_index("s")
    wid = sid * NC + cid
    base0 = wid * EPW

    if half == 0:
        init_src = zeros_hbm.at[pl.ds(sid * ROWS_PER_TILE, ROWS_PER_TILE)]
    else:
        init_src = zeros_hbm.at[cid, pl.ds(sid * ROWS_PER_TILE, ROWS_PER_TILE)]
    pltpu.sync_copy(
        init_src, acc.at[pl.ds(sid * ROWS_PER_TILE, ROWS_PER_TILE)])
    plsc.subcore_barrier()

    koff = half * K

    def idx_fire(i, q):
        pltpu.async_copy(
            idx_hbm.at[0, wid, koff + i], idxv.at[q, 0], i_sems[q])
        pltpu.async_copy(
            idx_hbm.at[1, wid, koff + i], idxv.at[q, 1], j_sems[q])

    def idx_wait(i, q):
        pltpu.make_async_copy(
            idx_hbm.at[0, wid, koff + i], idxv.at[q, 0], i_sems[q]).wait()
        pltpu.make_async_copy(
            idx_hbm.at[1, wid, koff + i], idxv.at[q, 1], j_sems[q]).wait()

    def g_fire(b, q):
        pltpu.async_copy(x1_hbm.at[idxv.at[q, 1]], gbufs[b], g_sems[b])

    def f_fire(i, fs):
        pltpu.async_copy(
            filt_hbm.at[pl.ds(base0 + i * C, 2 * C)], fbufs[fs], f_sems[fs])

    def f_wait(i, fs):
        pltpu.make_async_copy(
            filt_hbm.at[pl.ds(base0 + i * C, 2 * C)], fbufs[fs],
            f_sems[fs]).wait()

    def s_wait(b):
        pltpu.make_async_copy(gbufs[b], acc.at[idxv.at[0, 0]], s_sems[b]).wait()

    def unpack_words(w):
        lo = lax.bitcast_convert_type(jnp.left_shift(w, 16), jnp.float32)
        hi = lax.bitcast_convert_type(
            jnp.bitwise_and(w, jnp.int32(-65536)), jnp.float32)
        return lo, hi

    def proc(i, u):
        b = u % NBUF
        b2 = (u + PF) % NBUF
        q2 = (u + PF) % IRING
        q4 = (u + PF + 2) % IRING

        @pl.when(i + PF + 2 < K)
        def _():
            idx_fire(i + PF + 2, q4)

        @pl.when(i + PF < K)
        def _():
            @pl.when(i >= PF)
            def _():
                s_wait(b2)
            idx_wait(i + PF, q2)
            g_fire(b2, q2)
            if u % 2 == 0:
                f_fire(i + PF, ((u + PF) // 2) % FRING)

        pltpu.make_async_copy(
            x1_hbm.at[idxv.at[u, 1]], gbufs[b], g_sems[b]).wait()
        fs = (u // 2) % FRING
        if u % 2 == 0:
            f_wait(i, fs)

        gb, fb = gbufs[b], fbufs[fs]
        ro = (u % 2) * C

        def mul2(r, c2):
            for rr in range(2):
                rg = 2 * r + rr
                for j in range(D // 16):
                    s = pl.ds(j * 16, 16)
                    gb[rg, s] = gb[rg, s] * fb[ro + rg, s]
            return c2

        lax.fori_loop(0, C // 2, mul2, 0)
        pltpu.async_copy(gb, acc.at[idxv.at[u, 0]], s_sems[b], add=True)

    for j in range(PF + 2):
        idx_fire(j, j)
    f_fire(0, 0)
    for j in range(PF):
        idx_wait(j, j)
        g_fire(j % NBUF, j)

    def outer(i2, carry):
        for u in range(UNROLL):
            proc(i2 * UNROLL + u, u)
        return carry

    n_outer = K // UNROLL
    lax.fori_loop(0, n_outer, outer, 0)
    for i in range(n_outer * UNROLL, K):
        proc(jnp.int32(i), i % UNROLL)
    for b in range(NBUF):
        s_wait(b)

    plsc.subcore_barrier()

    pltpu.sync_copy(
        acc.at[pl.ds(sid * ROWS_PER_TILE, ROWS_PER_TILE)],
        out_hbm.at[cid, pl.ds(sid * ROWS_PER_TILE, ROWS_PER_TILE)],
    )


def _sc_gather_mul_scatter(x1, edge_idx, filt, zeros, half):
    mesh = plsc.VectorSubcoreMesh(core_axis_name="c", subcore_axis_name="s")
    f = functools.partial(
        pl.kernel,
        mesh=mesh,
        out_type=jax.ShapeDtypeStruct((NC, N_PAD, D), jnp.float32),
        scratch_types=[
            pltpu.VMEM((IRING, 2, C), jnp.int32),
            *[pltpu.VMEM((C, D), jnp.float32) for _ in range(NBUF)],
            *[pltpu.VMEM((2 * C, D), jnp.float32) for _ in range(FRING)],
            pltpu.VMEM_SHARED((N_PAD, D), jnp.float32),
            *[pltpu.SemaphoreType.DMA
              for _ in range(2 * NBUF + FRING + 2 * IRING)],
        ],
    )(_make_sc_body(half))
    return f(x1, edge_idx, filt, zeros)


def kernel(x, edge_index, edge_rbf, lin1_w, lin1_b, lin2_w, lin2_b,
           fn1_w, fn1_b, fn2_w, fn2_b):
    ei = edge_index.astype(jnp.int32)
    filt_a, x1, zeros = _filter_net(
        x, lin1_w, lin1_b, edge_rbf, fn1_w, fn1_b, fn2_w, fn2_b, 0)
    filt_b = _filter_net(
        x, lin1_w, lin1_b, edge_rbf, fn1_w, fn1_b, fn2_w, fn2_b, 1)
    idx4 = ei.reshape(2, NW, NH * K, C)
    pa = _sc_gather_mul_scatter(x1, idx4, filt_a, zeros, 0)
    pb = _sc_gather_mul_scatter(x1, idx4, filt_b, pa, 1)
    return _final_linear(pb, lin2_w, lin2_b)
